# Initial kernel scaffold; baseline (speedup 1.0000x reference)
import jax
import jax.numpy as jnp
from jax.experimental import pallas as pl


def kernel(x, edge_index, W1, b1, W2, b2, W3, b3, Wmu, bmu, Wls, bls):
    raise NotImplementedError("write your pallas kernel here")


# speedup vs baseline: 7.3833x; 7.3833x over previous
import functools

import jax
import jax.numpy as jnp
from jax import lax
from jax.experimental import pallas as pl
from jax.experimental.pallas import tpu as pltpu
from jax.experimental.pallas import tpu_sc as plsc

N_NODES = 10000
NP = 10240
NC = 2
NS = 16
NW = NC * NS
CHUNK = 128
ROWS_PER_TILE = NP // NS
BLK = 256
PANEL = 128


def _sc_mesh():
    return plsc.VectorSubcoreMesh(core_axis_name="c", subcore_axis_name="s")


@functools.lru_cache(maxsize=None)
def _make_deg_kernel(cpw):
    @functools.partial(
        pl.kernel,
        out_type=jax.ShapeDtypeStruct((NC, NP, PANEL), jnp.float32),
        mesh=_sc_mesh(),
        scratch_types=[
            pltpu.VMEM_SHARED((NP, PANEL), jnp.float32),
            pltpu.VMEM((cpw, CHUNK), jnp.int32),
            pltpu.VMEM((CHUNK, PANEL), jnp.float32),
        ],
    )
    def deg_kernel(dst_hbm, zeros_hbm, ones_hbm, out_hbm, acc, dsts, ones_v):
        c = lax.axis_index("c")
        s = lax.axis_index("s")
        wid = c * NS + s
        r0 = s * ROWS_PER_TILE
        pltpu.sync_copy(zeros_hbm, acc.at[pl.ds(r0, ROWS_PER_TILE)])
        pltpu.sync_copy(ones_hbm, ones_v)
        pltpu.sync_copy(dst_hbm.at[wid], dsts)
        plsc.subcore_barrier()

        def chunk(j, carry):
            pltpu.sync_copy(ones_v, acc.at[dsts.at[j]], add=True)
            return carry

        lax.fori_loop(0, cpw, chunk, 0)
        plsc.subcore_barrier()
        pltpu.sync_copy(acc.at[pl.ds(r0, ROWS_PER_TILE)],
                        out_hbm.at[c, pl.ds(r0, ROWS_PER_TILE)])

    return deg_kernel


@functools.lru_cache(maxsize=None)
def _make_agg_kernel(num_panels, cpw):
    out_type = [jax.ShapeDtypeStruct((NC, NP, PANEL), jnp.float32)
                for _ in range(num_panels)]

    @functools.partial(
        pl.kernel,
        out_type=out_type,
        mesh=_sc_mesh(),
        scratch_types=[
            pltpu.VMEM_SHARED((NP, PANEL), jnp.float32),
            pltpu.VMEM((cpw, CHUNK), jnp.int32),
            pltpu.VMEM((cpw, CHUNK), jnp.int32),
            pltpu.VMEM((CHUNK, PANEL), jnp.float32),
        ],
    )
    def agg_kernel(*refs):
        tables = refs[:num_panels]
        src_hbm, dst_hbm, zeros_hbm = refs[num_panels:num_panels + 3]
        outs = refs[num_panels + 3:2 * num_panels + 3]
        acc, srcs, dsts, gbuf = refs[2 * num_panels + 3:]

        c = lax.axis_index("c")
        s = lax.axis_index("s")
        wid = c * NS + s
        r0 = s * ROWS_PER_TILE
        pltpu.sync_copy(src_hbm.at[wid], srcs)
        pltpu.sync_copy(dst_hbm.at[wid], dsts)

        for p in range(num_panels):
            table = tables[p]

            @pl.when(c == 0)
            def _():
                pltpu.sync_copy(table.at[pl.ds(r0, ROWS_PER_TILE)],
                                acc.at[pl.ds(r0, ROWS_PER_TILE)])

            @pl.when(c != 0)
            def _():
                pltpu.sync_copy(zeros_hbm, acc.at[pl.ds(r0, ROWS_PER_TILE)])

            plsc.subcore_barrier()

            def chunk(j, carry):
                pltpu.sync_copy(table.at[srcs.at[j]], gbuf)
                pltpu.sync_copy(gbuf, acc.at[dsts.at[j]], add=True)
                return carry

            lax.fori_loop(0, cpw, chunk, 0)
            plsc.subcore_barrier()
            pltpu.sync_copy(acc.at[pl.ds(r0, ROWS_PER_TILE)],
                            outs[p].at[c, pl.ds(r0, ROWS_PER_TILE)])
            plsc.subcore_barrier()

    return agg_kernel


def _prescale_body(x_ref, dp_ref, o_ref, dis_ref):
    deg = dp_ref[0][:, 0:1] + dp_ref[1][:, 0:1] + 1.0
    dis = lax.rsqrt(jnp.maximum(deg, 1.0))
    o_ref[...] = x_ref[...] * dis
    dis_ref[...] = jnp.broadcast_to(dis, dis_ref.shape)


def _prescale(x_pad, degp):
    grid = (NP // BLK,)
    return pl.pallas_call(
        _prescale_body,
        grid=grid,
        in_specs=[
            pl.BlockSpec((BLK, x_pad.shape[1]), lambda i: (i, 0)),
            pl.BlockSpec((NC, BLK, PANEL), lambda i: (0, i, 0)),
        ],
        out_specs=[
            pl.BlockSpec((BLK, x_pad.shape[1]), lambda i: (i, 0)),
            pl.BlockSpec((BLK, PANEL), lambda i: (i, 0)),
        ],
        out_shape=[
            jax.ShapeDtypeStruct((NP, x_pad.shape[1]), jnp.float32),
            jax.ShapeDtypeStruct((NP, PANEL), jnp.float32),
        ],
    )(x_pad, degp)


def _layer_body(*refs, num_panels, relu, post):
    s_refs = refs[:num_panels]
    dis_ref, w_ref, b_ref, o_ref = refs[num_panels:]
    dis = dis_ref[:, 0:1]
    sacc = jnp.concatenate([r[0] + r[1] for r in s_refs], axis=1) * dis
    y = jnp.dot(sacc, w_ref[...], preferred_element_type=jnp.float32)
    y = y + b_ref[...]
    if relu:
        y = jnp.maximum(y, 0.0)
    if post:
        y = y * dis
    o_ref[...] = y


def _layer(parts, dis, W, b, relu=True, post=True):
    num_panels = len(parts)
    k, m = W.shape
    grid = (NP // BLK,)
    body = functools.partial(_layer_body, num_panels=num_panels,
                             relu=relu, post=post)
    in_specs = (
        [pl.BlockSpec((NC, BLK, PANEL), lambda i: (0, i, 0))] * num_panels
        + [pl.BlockSpec((BLK, PANEL), lambda i: (i, 0)),
           pl.BlockSpec((k, m), lambda i: (0, 0)),
           pl.BlockSpec((1, m), lambda i: (0, 0))]
    )
    return pl.pallas_call(
        body,
        grid=grid,
        in_specs=in_specs,
        out_specs=pl.BlockSpec((BLK, m), lambda i: (i, 0)),
        out_shape=jax.ShapeDtypeStruct((NP, m), jnp.float32),
    )(*parts, dis, W, b.reshape(1, m))


def _head_body(*refs, num_panels):
    s_refs = refs[:num_panels]
    dis_ref, w_ref, b_ref, wcat_ref, o_ref = refs[num_panels:]
    dis = dis_ref[:, 0:1]
    sacc = jnp.concatenate([r[0] + r[1] for r in s_refs], axis=1) * dis
    h = jnp.dot(sacc, w_ref[...], preferred_element_type=jnp.float32)
    h = jnp.maximum(h + b_ref[...], 0.0)
    o_ref[...] = jnp.dot(h, wcat_ref[...],
                         preferred_element_type=jnp.float32) * dis


def _head(parts, dis, W, b, Wcat):
    num_panels = len(parts)
    k, m = W.shape
    m2 = Wcat.shape[1]
    grid = (NP // BLK,)
    body = functools.partial(_head_body, num_panels=num_panels)
    in_specs = (
        [pl.BlockSpec((NC, BLK, PANEL), lambda i: (0, i, 0))] * num_panels
        + [pl.BlockSpec((BLK, PANEL), lambda i: (i, 0)),
           pl.BlockSpec((k, m), lambda i: (0, 0)),
           pl.BlockSpec((1, m), lambda i: (0, 0)),
           pl.BlockSpec((m, m2), lambda i: (0, 0))]
    )
    return pl.pallas_call(
        body,
        grid=grid,
        in_specs=in_specs,
        out_specs=pl.BlockSpec((BLK, m2), lambda i: (i, 0)),
        out_shape=jax.ShapeDtypeStruct((NP, m2), jnp.float32),
    )(*parts, dis, W, b.reshape(1, m), Wcat)


def _final_body(s_ref, dis_ref, b_ref, o_ref):
    dis = dis_ref[:, 0:1]
    o_ref[...] = (s_ref[0] + s_ref[1]) * dis + b_ref[...]


def _final(part, dis, bcat):
    m = bcat.shape[0]
    grid = (NP // BLK,)
    return pl.pallas_call(
        _final_body,
        grid=grid,
        in_specs=[
            pl.BlockSpec((NC, BLK, PANEL), lambda i: (0, i, 0)),
            pl.BlockSpec((BLK, PANEL), lambda i: (i, 0)),
            pl.BlockSpec((1, m), lambda i: (0, 0)),
        ],
        out_specs=pl.BlockSpec((BLK, m), lambda i: (i, 0)),
        out_shape=jax.ShapeDtypeStruct((NP, m), jnp.float32),
    )(part, dis, bcat.reshape(1, m))


def _panels(h):
    return [h[:, p * PANEL:(p + 1) * PANEL] for p in range(h.shape[1] // PANEL)]


def kernel(x, edge_index, W1, b1, W2, b2, W3, b3, Wmu, bmu, Wls, bls):
    n, in_dim = x.shape
    e = edge_index.shape[1]
    cpw = -(-e // (NW * CHUNK))
    ep = NW * cpw * CHUNK

    ei = edge_index.astype(jnp.int32)
    pad_idx = jnp.full((ep - e,), N_NODES, jnp.int32)
    src3 = jnp.concatenate([ei[0], pad_idx]).reshape(NW, cpw, CHUNK)
    dst3 = jnp.concatenate([ei[1], pad_idx]).reshape(NW, cpw, CHUNK)
    x_pad = jnp.pad(x, ((0, NP - n), (0, 0)))

    zeros128 = jnp.zeros((ROWS_PER_TILE, PANEL), jnp.float32)
    ones128 = jnp.ones((CHUNK, PANEL), jnp.float32)

    degp = _make_deg_kernel(cpw)(dst3, zeros128, ones128)

    xs, dis = _prescale(x_pad, degp)

    s1 = _make_agg_kernel(1, cpw)(xs, src3, dst3, zeros128)
    h2 = _layer(s1, dis, W1, b1)

    s2 = _make_agg_kernel(2, cpw)(*_panels(h2), src3, dst3, zeros128)
    h3 = _layer(s2, dis, W2, b2)

    s3 = _make_agg_kernel(4, cpw)(*_panels(h3), src3, dst3, zeros128)
    Wcat = jnp.concatenate([Wmu, Wls], axis=1)
    g = _head(s3, dis, W3, b3, Wcat)

    s4 = _make_agg_kernel(1, cpw)(g, src3, dst3, zeros128)
    bcat = jnp.concatenate([bmu, bls])
    out = _final(s4[0], dis, bcat)

    half = Wmu.shape[1]
    return out[:n, :half], out[:n, half:2 * half]
